# baseline (speedup 1.0000x reference)
import functools

import jax
import jax.numpy as jnp
from jax import lax
from jax.experimental import pallas as pl
from jax.experimental.pallas import tpu as pltpu
from jax.experimental.pallas import tpu_sc as plsc

VOCAB = 100000
EMB = 16
BATCH = 1024
CTX = 20

_NC = 2
_NS = 16
_NW = _NC * _NS
_S_PER_W = BATCH // _NW
_IDX_PER_W = _S_PER_W * CTX
_GCHUNK = 128
_NCHUNK = _IDX_PER_W // _GCHUNK
_RPP = 128 // EMB
_ACC_ROWS = _S_PER_W * _RPP


@functools.cache
def _make_sc_embed_mean():
    mesh = plsc.VectorSubcoreMesh(core_axis_name="c", subcore_axis_name="s")

    @functools.partial(
        pl.kernel,
        mesh=mesh,
        out_type=jax.ShapeDtypeStruct((BATCH * EMB,), jnp.float32),
        scratch_types=[
            pltpu.VMEM((_NCHUNK, _GCHUNK), jnp.int32),
            pltpu.VMEM((_NCHUNK, _GCHUNK), jnp.int32),
            pltpu.VMEM((_ACC_ROWS, 128), jnp.float32),
            pltpu.VMEM_SHARED((_NS * _ACC_ROWS, 128), jnp.float32),
            pltpu.VMEM((_S_PER_W * EMB,), jnp.float32),
            pltpu.SemaphoreType.DMA,
            pltpu.SemaphoreType.DMA,
        ],
    )
    def sc_embed_mean(idxd_hbm, sidx_hbm, table_hbm, out_hbm, idxd_v, sidx_v,
                      rows_v, acc_sh, avg_v, gsem, ssem):
        wid = lax.axis_index("s") * _NC + lax.axis_index("c")
        sid = lax.axis_index("s")
        abase = sid * _ACC_ROWS
        pltpu.sync_copy(idxd_hbm.at[wid], idxd_v)
        pltpu.sync_copy(sidx_hbm.at[wid], sidx_v)

        zeros16 = jnp.zeros((16,), jnp.float32)
        for s in range(_S_PER_W):
            for m in range(_RPP):
                rows_v[s * _RPP + m, pl.ds(m * EMB, EMB)] = zeros16
        pltpu.sync_copy(rows_v, acc_sh.at[pl.ds(abase, _ACC_ROWS)])

        for k in range(_NCHUNK):
            half = pl.ds((k % 2) * _GCHUNK, _GCHUNK)
            pltpu.async_copy(
                table_hbm.at[idxd_v.at[k]], rows_v.at[half], gsem,
            ).wait()
            pltpu.async_copy(
                rows_v.at[half], acc_sh.at[sidx_v.at[k]], ssem, add=True,
            ).wait()

        pltpu.sync_copy(acc_sh.at[pl.ds(abase, _ACC_ROWS)], rows_v)
        inv = jnp.float32(1.0 / CTX)
        for s in range(_S_PER_W):
            acc = rows_v[s * _RPP + 0, pl.ds(0, EMB)]
            for m in range(1, _RPP):
                acc = acc + rows_v[s * _RPP + m, pl.ds(m * EMB, EMB)]
            avg_v[pl.ds(s * EMB, EMB)] = acc * inv

        pltpu.sync_copy(
            avg_v,
            out_hbm.at[pl.ds(wid * _S_PER_W * EMB, _S_PER_W * EMB)])

    return sc_embed_mean


_VT = 2048
_GRID = (VOCAB + _VT - 1) // _VT


def _proj_body(avg_ref, w_ref, b_ref, out_ref):
    out_ref[...] = (
        jnp.dot(avg_ref[...], w_ref[...], preferred_element_type=jnp.float32)
        + b_ref[...]
    )


def _tc_project(avg, W, b2d):
    return pl.pallas_call(
        _proj_body,
        grid=(_GRID,),
        in_specs=[
            pl.BlockSpec((BATCH, EMB), lambda j: (0, 0)),
            pl.BlockSpec((EMB, _VT), lambda j: (0, j)),
            pl.BlockSpec((1, _VT), lambda j: (0, j)),
        ],
        out_specs=pl.BlockSpec((BATCH, _VT), lambda j: (0, j)),
        out_shape=jax.ShapeDtypeStruct((BATCH, VOCAB), jnp.float32),
    )(avg, W, b2d)


def kernel(inputs, emb_table, W, b):
    idx = inputs.reshape(-1).astype(jnp.int32)
    idxd = lax.shift_right_logical(idx, 3).reshape(_NW, _NCHUNK, _GCHUNK)
    tok = jnp.arange(BATCH * CTX, dtype=jnp.int32)
    wid_of_tok = tok // _IDX_PER_W
    local_s = (tok - wid_of_tok * _IDX_PER_W) // CTX
    sidx = ((wid_of_tok // _NC) * _ACC_ROWS + local_s * _RPP
            + (idx & (_RPP - 1))).reshape(_NW, _NCHUNK, _GCHUNK)
    table128 = emb_table.reshape(VOCAB // _RPP, 128)
    avg = _make_sc_embed_mean()(idxd, sidx, table128).reshape(BATCH, EMB)
    return _tc_project(avg, W, b.reshape(1, VOCAB))


# speedup vs baseline: 1.0045x; 1.0045x over previous
import functools

import jax
import jax.numpy as jnp
from jax import lax
from jax.experimental import pallas as pl
from jax.experimental.pallas import tpu as pltpu
from jax.experimental.pallas import tpu_sc as plsc

VOCAB = 100000
EMB = 16
BATCH = 1024
CTX = 20

_NC = 2
_NS = 16
_NW = _NC * _NS
_S_PER_W = BATCH // _NW
_IDX_PER_W = _S_PER_W * CTX
_GCHUNK = 128
_NCHUNK = _IDX_PER_W // _GCHUNK
_RPP = 128 // EMB
_ACC_ROWS = _S_PER_W * _RPP


@functools.cache
def _make_sc_embed_mean():
    mesh = plsc.VectorSubcoreMesh(core_axis_name="c", subcore_axis_name="s")

    @functools.partial(
        pl.kernel,
        mesh=mesh,
        out_type=jax.ShapeDtypeStruct((BATCH * EMB,), jnp.float32),
        scratch_types=[
            pltpu.VMEM((_NCHUNK, _GCHUNK), jnp.int32),
            pltpu.VMEM((_NCHUNK, _GCHUNK), jnp.int32),
            pltpu.VMEM((_ACC_ROWS, 128), jnp.float32),
            pltpu.VMEM_SHARED((_NS * _ACC_ROWS, 128), jnp.float32),
            pltpu.VMEM((_S_PER_W * EMB,), jnp.float32),
            pltpu.SemaphoreType.DMA,
            pltpu.SemaphoreType.DMA,
        ],
    )
    def sc_embed_mean(idxd_hbm, sidx_hbm, table_hbm, out_hbm, idxd_v, sidx_v,
                      rows_v, acc_sh, avg_v, gsem, ssem):
        wid = lax.axis_index("s") * _NC + lax.axis_index("c")
        sid = lax.axis_index("s")
        abase = sid * _ACC_ROWS
        pltpu.sync_copy(idxd_hbm.at[wid], idxd_v)
        pltpu.sync_copy(sidx_hbm.at[wid], sidx_v)

        zeros16 = jnp.zeros((16,), jnp.float32)
        for s in range(_S_PER_W):
            for m in range(_RPP):
                rows_v[s * _RPP + m, pl.ds(m * EMB, EMB)] = zeros16
        pltpu.sync_copy(rows_v, acc_sh.at[pl.ds(abase, _ACC_ROWS)])

        for k in range(_NCHUNK):
            half = pl.ds((k % 2) * _GCHUNK, _GCHUNK)
            pltpu.async_copy(
                table_hbm.at[idxd_v.at[k]], rows_v.at[half], gsem,
            ).wait()
            pltpu.async_copy(
                rows_v.at[half], acc_sh.at[sidx_v.at[k]], ssem, add=True,
            ).wait()

        pltpu.sync_copy(acc_sh.at[pl.ds(abase, _ACC_ROWS)], rows_v)
        inv = jnp.float32(1.0 / CTX)
        for s in range(_S_PER_W):
            acc = rows_v[s * _RPP + 0, pl.ds(0, EMB)]
            for m in range(1, _RPP):
                acc = acc + rows_v[s * _RPP + m, pl.ds(m * EMB, EMB)]
            avg_v[pl.ds(s * EMB, EMB)] = acc * inv

        pltpu.sync_copy(
            avg_v,
            out_hbm.at[pl.ds(wid * _S_PER_W * EMB, _S_PER_W * EMB)])

    return sc_embed_mean


_BT = 16
_GRID = BATCH // _BT


def _proj_body(avg_ref, w_ref, b_ref, out_ref):
    out_ref[...] = (
        jnp.dot(avg_ref[...], w_ref[...], preferred_element_type=jnp.float32)
        + b_ref[...]
    )


def _tc_project(avg, W, b2d):
    return pl.pallas_call(
        _proj_body,
        grid=(_GRID,),
        in_specs=[
            pl.BlockSpec((_BT, EMB), lambda i: (i, 0)),
            pl.BlockSpec((EMB, VOCAB), lambda i: (0, 0)),
            pl.BlockSpec((1, VOCAB), lambda i: (0, 0)),
        ],
        out_specs=pl.BlockSpec((_BT, VOCAB), lambda i: (i, 0)),
        out_shape=jax.ShapeDtypeStruct((BATCH, VOCAB), jnp.float32),
    )(avg, W, b2d)


def kernel(inputs, emb_table, W, b):
    idx = inputs.reshape(-1).astype(jnp.int32)
    idxd = lax.shift_right_logical(idx, 3).reshape(_NW, _NCHUNK, _GCHUNK)
    tok = jnp.arange(BATCH * CTX, dtype=jnp.int32)
    wid_of_tok = tok // _IDX_PER_W
    local_s = (tok - wid_of_tok * _IDX_PER_W) // CTX
    sidx = ((wid_of_tok // _NC) * _ACC_ROWS + local_s * _RPP
            + (idx & (_RPP - 1))).reshape(_NW, _NCHUNK, _GCHUNK)
    table128 = emb_table.reshape(VOCAB // _RPP, 128)
    avg = _make_sc_embed_mean()(idxd, sidx, table128).reshape(BATCH, EMB)
    return _tc_project(avg, W, b.reshape(1, VOCAB))


# speedup vs baseline: 2.2354x; 2.2255x over previous
import functools

import jax
import jax.numpy as jnp
from jax import lax
from jax.experimental import pallas as pl
from jax.experimental.pallas import tpu as pltpu
from jax.experimental.pallas import tpu_sc as plsc

VOCAB = 100000
EMB = 16
BATCH = 1024
CTX = 20

_NC = 2
_NS = 16
_NW = _NC * _NS
_S_PER_W = BATCH // _NW
_IDX_PER_W = _S_PER_W * CTX
_GCHUNK = 128
_NCHUNK = _IDX_PER_W // _GCHUNK
_RPP = 128 // EMB
_ACC_ROWS = _S_PER_W * _RPP


@functools.cache
def _make_sc_embed_mean():
    mesh = plsc.VectorSubcoreMesh(core_axis_name="c", subcore_axis_name="s")

    @functools.partial(
        pl.kernel,
        mesh=mesh,
        out_type=jax.ShapeDtypeStruct((BATCH * EMB,), jnp.float32),
        scratch_types=[
            pltpu.VMEM((_NCHUNK, _GCHUNK), jnp.int32),
            pltpu.VMEM((_NCHUNK, _GCHUNK), jnp.int32),
            pltpu.VMEM((_ACC_ROWS, 128), jnp.float32),
            pltpu.VMEM_SHARED((_NS * _ACC_ROWS, 128), jnp.float32),
            pltpu.VMEM((_S_PER_W * EMB,), jnp.float32),
            pltpu.SemaphoreType.DMA,
            pltpu.SemaphoreType.DMA,
        ],
    )
    def sc_embed_mean(idxd_hbm, sidx_hbm, table_hbm, out_hbm, idxd_v, sidx_v,
                      rows_v, acc_sh, avg_v, gsem, ssem):
        wid = lax.axis_index("s") * _NC + lax.axis_index("c")
        sid = lax.axis_index("s")
        abase = sid * _ACC_ROWS
        pltpu.sync_copy(idxd_hbm.at[wid], idxd_v)
        pltpu.sync_copy(sidx_hbm.at[wid], sidx_v)

        zeros16 = jnp.zeros((16,), jnp.float32)
        for s in range(_S_PER_W):
            for m in range(_RPP):
                rows_v[s * _RPP + m, pl.ds(m * EMB, EMB)] = zeros16
        pltpu.sync_copy(rows_v, acc_sh.at[pl.ds(abase, _ACC_ROWS)])

        for k in range(_NCHUNK):
            half = pl.ds((k % 2) * _GCHUNK, _GCHUNK)
            pltpu.async_copy(
                table_hbm.at[idxd_v.at[k]], rows_v.at[half], gsem,
            ).wait()
            pltpu.async_copy(
                rows_v.at[half], acc_sh.at[sidx_v.at[k]], ssem, add=True,
            ).wait()

        pltpu.sync_copy(acc_sh.at[pl.ds(abase, _ACC_ROWS)], rows_v)
        inv = jnp.float32(1.0 / CTX)
        for s in range(_S_PER_W):
            acc = rows_v[s * _RPP + 0, pl.ds(0, EMB)]
            for m in range(1, _RPP):
                acc = acc + rows_v[s * _RPP + m, pl.ds(m * EMB, EMB)]
            avg_v[pl.ds(s * EMB, EMB)] = acc * inv

        pltpu.sync_copy(
            avg_v,
            out_hbm.at[pl.ds(wid * _S_PER_W * EMB, _S_PER_W * EMB)])

    return sc_embed_mean


_VT = 2048
_GRID = (VOCAB + _VT - 1) // _VT


def _proj_body(w_ref, avg_ref, b_ref, out_ref):
    out_ref[...] = (
        lax.dot_general(
            w_ref[...], avg_ref[...],
            dimension_numbers=(((0,), (1,)), ((), ())),
            preferred_element_type=jnp.float32,
        )
        + b_ref[...]
    )


def _tc_project_t(avg, W, bcol):
    return pl.pallas_call(
        _proj_body,
        grid=(_GRID,),
        in_specs=[
            pl.BlockSpec((EMB, _VT), lambda j: (0, j)),
            pl.BlockSpec((BATCH, EMB), lambda j: (0, 0)),
            pl.BlockSpec((_VT, 1), lambda j: (j, 0)),
        ],
        out_specs=pl.BlockSpec((_VT, BATCH), lambda j: (j, 0)),
        out_shape=jax.ShapeDtypeStruct((VOCAB, BATCH), jnp.float32),
    )(W, avg, bcol)


def kernel(inputs, emb_table, W, b):
    idx = inputs.reshape(-1).astype(jnp.int32)
    idxd = lax.shift_right_logical(idx, 3).reshape(_NW, _NCHUNK, _GCHUNK)
    tok = jnp.arange(BATCH * CTX, dtype=jnp.int32)
    wid_of_tok = tok // _IDX_PER_W
    local_s = (tok - wid_of_tok * _IDX_PER_W) // CTX
    sidx = ((wid_of_tok // _NC) * _ACC_ROWS + local_s * _RPP
            + (idx & (_RPP - 1))).reshape(_NW, _NCHUNK, _GCHUNK)
    table128 = emb_table.reshape(VOCAB // _RPP, 128)
    avg = _make_sc_embed_mean()(idxd, sidx, table128).reshape(BATCH, EMB)
    return _tc_project_t(avg, W, b.reshape(VOCAB, 1)).T
